# baseline (speedup 1.0000x reference)
import functools

import jax
import jax.numpy as jnp
from jax import lax
from jax.experimental import pallas as pl
from jax.experimental.pallas import tpu as pltpu
from jax.experimental.pallas import tpu_sc as plsc

F32 = jnp.float32


def _mm(a, b):
    return lax.dot_general(a, b, (((1,), (0,)), ((), ())),
                           preferred_element_type=F32)


def _k1_body(p_ref, w_ref, b_ref, o_ref):
    y = _mm(p_ref[0], w_ref[...]) + b_ref[...]
    o_ref[0] = jnp.maximum(y, 0.0)


def _conv1(patches, w, b):
    B, N, K = patches.shape
    G = 4
    Nb = N // G
    return pl.pallas_call(
        _k1_body,
        grid=(B, G),
        in_specs=[pl.BlockSpec((1, Nb, K), lambda bb, g: (bb, g, 0)),
                  pl.BlockSpec((K, 128), lambda bb, g: (0, 0)),
                  pl.BlockSpec((1, 128), lambda bb, g: (0, 0))],
        out_specs=pl.BlockSpec((1, Nb, 128), lambda bb, g: (bb, g, 0)),
        out_shape=jax.ShapeDtypeStruct((B, N, 128), F32),
    )(patches, w, b)


def _k2_body(p00_ref, p01_ref, p10_ref, p11_ref, w_ref, b_ref, o_ref):
    g = pl.program_id(1)
    P = ((p00_ref, p01_ref), (p10_ref, p11_ref))
    acc = jnp.zeros((14 * 56, 256), F32)
    for kh in range(4):
        a, r0 = kh % 2, kh // 2
        for kw in range(4):
            c, c0 = kw % 2, kw // 2
            slab = P[a][c][0, pl.ds(g * 14 + r0, 14), pl.ds(c0, 56), :]
            acc += _mm(slab.reshape(14 * 56, 128), w_ref[kh * 4 + kw])
    o_ref[0] = jnp.maximum(acc + b_ref[...], 0.0).reshape(14, 56, 256)


def _conv2(p00, p01, p10, p11, w, b):
    return pl.pallas_call(
        _k2_body,
        grid=(8, 4),
        in_specs=[pl.BlockSpec((1, 57, 57, 128), lambda bb, g: (bb, 0, 0, 0))] * 4
        + [pl.BlockSpec((16, 128, 256), lambda bb, g: (0, 0, 0)),
           pl.BlockSpec((1, 256), lambda bb, g: (0, 0))],
        out_specs=pl.BlockSpec((1, 14, 56, 256), lambda bb, g: (bb, g, 0, 0)),
        out_shape=jax.ShapeDtypeStruct((8, 56, 56, 256), F32),
    )(p00, p01, p10, p11, w, b)


def _k3_body(x_ref, w1_ref, w2_ref, o_ref):
    g = pl.program_id(1)
    acc = jnp.zeros((784, 256), F32)
    for kh in range(3):
        for kw in range(3):
            slab = x_ref[0, pl.ds(g * 14 + kh, 14), pl.ds(kw, 56), :]
            acc += _mm(jnp.maximum(slab, 0.0).reshape(784, 256),
                       w1_ref[kh * 3 + kw])
    r = _mm(jnp.maximum(acc, 0.0), w2_ref[...])
    ctr = x_ref[0, pl.ds(g * 14 + 1, 14), pl.ds(1, 56), :].reshape(784, 256)
    o_ref[0] = (ctr + r).reshape(14, 56, 256)


def _resblock(xpad, w1, w2):
    return pl.pallas_call(
        _k3_body,
        grid=(8, 4),
        in_specs=[pl.BlockSpec((1, 58, 58, 256), lambda bb, g: (bb, 0, 0, 0)),
                  pl.BlockSpec((9, 256, 256), lambda bb, g: (0, 0, 0)),
                  pl.BlockSpec((256, 256), lambda bb, g: (0, 0))],
        out_specs=pl.BlockSpec((1, 14, 56, 256), lambda bb, g: (bb, g, 0, 0)),
        out_shape=jax.ShapeDtypeStruct((8, 56, 56, 256), F32),
    )(xpad, w1, w2)


def _k4_body(x_ref, w1_ref, w2_ref, wo_ref, bo_ref, embT_ref, e2_ref,
             lat_ref, idx_ref):
    g = pl.program_id(1)
    acc = jnp.zeros((784, 256), F32)
    for kh in range(3):
        for kw in range(3):
            slab = x_ref[0, pl.ds(g * 14 + kh, 14), pl.ds(kw, 56), :]
            acc += _mm(jnp.maximum(slab, 0.0).reshape(784, 256),
                       w1_ref[kh * 3 + kw])
    r = _mm(jnp.maximum(acc, 0.0), w2_ref[...])
    ctr = x_ref[0, pl.ds(g * 14 + 1, 14), pl.ds(1, 56), :].reshape(784, 256)
    h = ctr + r
    f = _mm(jnp.maximum(h, 0.0), wo_ref[...]) + bo_ref[...]
    lat_ref[0] = f
    f2 = jnp.sum(f * f, axis=1, keepdims=True)
    s = (f2 + e2_ref[...]) - 2.0 * _mm(f, embT_ref[...])
    m = jnp.min(s, axis=1, keepdims=True)
    iota = lax.broadcasted_iota(jnp.int32, (784, 512), 1)
    idxv = jnp.min(jnp.where(s == m, iota, 512), axis=1)
    idx_ref[0, 0, 0] = idxv


def _res_proj_vq(xpad, w1, w2, wo, bo, embT, e2):
    return pl.pallas_call(
        _k4_body,
        grid=(8, 4),
        in_specs=[pl.BlockSpec((1, 58, 58, 256), lambda bb, g: (bb, 0, 0, 0)),
                  pl.BlockSpec((9, 256, 256), lambda bb, g: (0, 0, 0)),
                  pl.BlockSpec((256, 256), lambda bb, g: (0, 0)),
                  pl.BlockSpec((256, 64), lambda bb, g: (0, 0)),
                  pl.BlockSpec((1, 64), lambda bb, g: (0, 0)),
                  pl.BlockSpec((64, 512), lambda bb, g: (0, 0)),
                  pl.BlockSpec((1, 512), lambda bb, g: (0, 0))],
        out_specs=[pl.BlockSpec((1, 784, 64), lambda bb, g: (bb, g, 0)),
                   pl.BlockSpec((1, 1, 1, 784), lambda bb, g: (bb, g, 0, 0))],
        out_shape=[jax.ShapeDtypeStruct((8, 3136, 64), F32),
                   jax.ShapeDtypeStruct((8, 4, 1, 784), jnp.int32)],
    )(xpad, w1, w2, wo, bo, embT, e2)


_NROWS = 25088
_NW = 32
_BPW = _NROWS // _NW


def _sc_gather_body(table_hbm, idx_hbm, out_hbm, idx_v, rows_v, sem):
    info = plsc.get_sparse_core_info()
    wid = lax.axis_index("s") * info.num_cores + lax.axis_index("c")
    base = wid * _BPW
    pltpu.sync_copy(idx_hbm.at[pl.ds(base, _BPW)], idx_v)
    pltpu.async_copy(table_hbm.at[idx_v], rows_v, sem).wait()
    pltpu.sync_copy(rows_v, out_hbm.at[pl.ds(base, _BPW)])


def _sc_gather(table, idx_flat):
    mesh = plsc.VectorSubcoreMesh(core_axis_name="c", subcore_axis_name="s")
    f = functools.partial(
        pl.kernel,
        mesh=mesh,
        out_type=jax.ShapeDtypeStruct((_NROWS, 128), F32),
        scratch_types=[pltpu.VMEM((_BPW,), jnp.int32),
                       pltpu.VMEM((_BPW, 128), F32),
                       pltpu.SemaphoreType.DMA],
    )(_sc_gather_body)
    return f(table, idx_flat)


def kernel(x, params):
    B = x.shape[0]
    xn = jnp.transpose(x, (0, 2, 3, 1))
    xp = jnp.pad(xn, ((0, 0), (1, 1), (1, 1), (0, 0)))

    taps = [xp[:, kh:kh + 224:2, kw:kw + 224:2, :]
            for kh in range(4) for kw in range(4)]
    patches = jnp.concatenate(taps, axis=-1).reshape(B, 112 * 112, 48)
    w1 = jnp.transpose(params['down_w'][0], (2, 3, 1, 0)).reshape(48, 128)
    b1 = params['down_b'][0].reshape(1, 128)
    h1 = _conv1(patches, w1, b1).reshape(B, 112, 112, 128)

    h1p = jnp.pad(h1, ((0, 0), (1, 1), (1, 1), (0, 0)))
    p00 = h1p[:, 0::2, 0::2, :]
    p01 = h1p[:, 0::2, 1::2, :]
    p10 = h1p[:, 1::2, 0::2, :]
    p11 = h1p[:, 1::2, 1::2, :]
    w2 = jnp.transpose(params['down_w'][1], (2, 3, 1, 0)).reshape(16, 128, 256)
    b2 = params['down_b'][1].reshape(1, 256)
    h2 = _conv2(p00, p01, p10, p11, w2, b2)

    rw1 = [jnp.transpose(w, (2, 3, 1, 0)).reshape(9, 256, 256)
           for w in params['res_w1']]
    rw2 = [jnp.transpose(w[:, :, 0, 0], (1, 0)) for w in params['res_w2']]
    h2p = jnp.pad(h2, ((0, 0), (1, 1), (1, 1), (0, 0)))
    h3 = _resblock(h2p, rw1[0], rw2[0])

    emb = params['codebook']
    wo = jnp.transpose(params['out_w'][:, :, 0, 0], (1, 0))
    bo = params['out_b'].reshape(1, 64)
    embT = jnp.transpose(emb, (1, 0))
    e2 = jnp.sum(emb * emb, axis=1).reshape(1, 512)
    h3p = jnp.pad(h3, ((0, 0), (1, 1), (1, 1), (0, 0)))
    lat, idx = _res_proj_vq(h3p, rw1[1], rw2[1], wo, bo, embT, e2)

    emb_pad = jnp.pad(emb, ((0, 0), (0, 64)))
    q = _sc_gather(emb_pad, idx.reshape(_NROWS))[:, :64].reshape(B, 3136, 64)

    latent = jnp.transpose(lat.reshape(B, 56, 56, 64), (0, 3, 1, 2))
    quantized = jnp.transpose(q.reshape(B, 56, 56, 64), (0, 3, 1, 2))
    qwg = latent + lax.stop_gradient(quantized - latent)
    indices = idx.reshape(B, 56, 56)
    return latent, qwg, quantized, indices


# speedup vs baseline: 1.0013x; 1.0013x over previous
import functools

import jax
import jax.numpy as jnp
from jax import lax
from jax.experimental import pallas as pl
from jax.experimental.pallas import tpu as pltpu
from jax.experimental.pallas import tpu_sc as plsc

F32 = jnp.float32


def _mm(a, b):
    return lax.dot_general(a, b, (((1,), (0,)), ((), ())),
                           preferred_element_type=F32)


def _k1_body(p_ref, w_ref, b_ref, o_ref):
    y = _mm(p_ref[0], w_ref[...]) + b_ref[...]
    o_ref[0] = jnp.maximum(y, 0.0)


def _conv1(patches, w, b):
    B, N, K = patches.shape
    G = 4
    Nb = N // G
    return pl.pallas_call(
        _k1_body,
        grid=(B, G),
        in_specs=[pl.BlockSpec((1, Nb, K), lambda bb, g: (bb, g, 0)),
                  pl.BlockSpec((K, 128), lambda bb, g: (0, 0)),
                  pl.BlockSpec((1, 128), lambda bb, g: (0, 0))],
        out_specs=pl.BlockSpec((1, Nb, 128), lambda bb, g: (bb, g, 0)),
        out_shape=jax.ShapeDtypeStruct((B, N, 128), F32),
    )(patches, w, b)


def _k2_body(p00_ref, p01_ref, p10_ref, p11_ref, w_ref, b_ref, o_ref):
    g = pl.program_id(1)
    P = ((p00_ref, p01_ref), (p10_ref, p11_ref))
    acc = jnp.zeros((14 * 56, 256), F32)
    for kh in range(4):
        a, r0 = kh % 2, kh // 2
        for kw in range(4):
            c, c0 = kw % 2, kw // 2
            slab = P[a][c][0, pl.ds(g * 14 + r0, 14), pl.ds(c0, 56), :]
            acc += _mm(slab.reshape(14 * 56, 128), w_ref[kh * 4 + kw])
    o_ref[0] = jnp.maximum(acc + b_ref[...], 0.0).reshape(14, 56, 256)


def _conv2(p00, p01, p10, p11, w, b):
    return pl.pallas_call(
        _k2_body,
        grid=(8, 4),
        in_specs=[pl.BlockSpec((1, 57, 57, 128), lambda bb, g: (bb, 0, 0, 0))] * 4
        + [pl.BlockSpec((16, 128, 256), lambda bb, g: (0, 0, 0)),
           pl.BlockSpec((1, 256), lambda bb, g: (0, 0))],
        out_specs=pl.BlockSpec((1, 14, 56, 256), lambda bb, g: (bb, g, 0, 0)),
        out_shape=jax.ShapeDtypeStruct((8, 56, 56, 256), F32),
    )(p00, p01, p10, p11, w, b)


def _k3_body(x_ref, w1_ref, w2_ref, o_ref):
    g = pl.program_id(1)
    acc = jnp.zeros((784, 256), F32)
    for kh in range(3):
        for kw in range(3):
            slab = x_ref[0, pl.ds(g * 14 + kh, 14), pl.ds(kw, 56), :]
            acc += _mm(jnp.maximum(slab, 0.0).reshape(784, 256),
                       w1_ref[kh * 3 + kw])
    r = _mm(jnp.maximum(acc, 0.0), w2_ref[...])
    ctr = x_ref[0, pl.ds(g * 14 + 1, 14), pl.ds(1, 56), :].reshape(784, 256)
    o_ref[0] = (ctr + r).reshape(14, 56, 256)


def _resblock(xpad, w1, w2):
    return pl.pallas_call(
        _k3_body,
        grid=(8, 4),
        in_specs=[pl.BlockSpec((1, 58, 58, 256), lambda bb, g: (bb, 0, 0, 0)),
                  pl.BlockSpec((9, 256, 256), lambda bb, g: (0, 0, 0)),
                  pl.BlockSpec((256, 256), lambda bb, g: (0, 0))],
        out_specs=pl.BlockSpec((1, 14, 56, 256), lambda bb, g: (bb, g, 0, 0)),
        out_shape=jax.ShapeDtypeStruct((8, 56, 56, 256), F32),
    )(xpad, w1, w2)


def _k4_body(x_ref, w1_ref, w2_ref, wo_ref, bo_ref, embT_ref, e2_ref,
             lat_ref, idx_ref):
    g = pl.program_id(1)
    acc = jnp.zeros((784, 256), F32)
    for kh in range(3):
        for kw in range(3):
            slab = x_ref[0, pl.ds(g * 14 + kh, 14), pl.ds(kw, 56), :]
            acc += _mm(jnp.maximum(slab, 0.0).reshape(784, 256),
                       w1_ref[kh * 3 + kw])
    r = _mm(jnp.maximum(acc, 0.0), w2_ref[...])
    ctr = x_ref[0, pl.ds(g * 14 + 1, 14), pl.ds(1, 56), :].reshape(784, 256)
    h = ctr + r
    f = _mm(jnp.maximum(h, 0.0), wo_ref[...]) + bo_ref[...]
    lat_ref[0] = f
    f2 = jnp.sum(f * f, axis=1, keepdims=True)
    s = (f2 + e2_ref[...]) - 2.0 * _mm(f, embT_ref[...])
    m = jnp.min(s, axis=1, keepdims=True)
    iota = lax.broadcasted_iota(jnp.int32, (784, 512), 1)
    idxv = jnp.min(jnp.where(s == m, iota, 512), axis=1)
    idx_ref[0, 0, 0] = idxv


def _res_proj_vq(xpad, w1, w2, wo, bo, embT, e2):
    return pl.pallas_call(
        _k4_body,
        grid=(8, 4),
        in_specs=[pl.BlockSpec((1, 58, 58, 256), lambda bb, g: (bb, 0, 0, 0)),
                  pl.BlockSpec((9, 256, 256), lambda bb, g: (0, 0, 0)),
                  pl.BlockSpec((256, 256), lambda bb, g: (0, 0)),
                  pl.BlockSpec((256, 64), lambda bb, g: (0, 0)),
                  pl.BlockSpec((1, 64), lambda bb, g: (0, 0)),
                  pl.BlockSpec((64, 512), lambda bb, g: (0, 0)),
                  pl.BlockSpec((1, 512), lambda bb, g: (0, 0))],
        out_specs=[pl.BlockSpec((1, 784, 64), lambda bb, g: (bb, g, 0)),
                   pl.BlockSpec((1, 1, 1, 784), lambda bb, g: (bb, g, 0, 0))],
        out_shape=[jax.ShapeDtypeStruct((8, 3136, 64), F32),
                   jax.ShapeDtypeStruct((8, 4, 1, 784), jnp.int32)],
    )(xpad, w1, w2, wo, bo, embT, e2)


_NROWS = 25088
_NW = 32
_BPW = _NROWS // _NW


def _sc_gather_body(table_hbm, idx_hbm, out_hbm, idx_v, rows_v, sem):
    info = plsc.get_sparse_core_info()
    wid = lax.axis_index("s") * info.num_cores + lax.axis_index("c")
    base = wid * _BPW
    pltpu.sync_copy(idx_hbm.at[pl.ds(base, _BPW)], idx_v)
    pltpu.async_copy(table_hbm.at[idx_v], rows_v, sem).wait()
    pltpu.sync_copy(rows_v, out_hbm.at[pl.ds(base, _BPW)])


def _sc_gather(table3, idx_flat):
    mesh = plsc.VectorSubcoreMesh(core_axis_name="c", subcore_axis_name="s")
    f = functools.partial(
        pl.kernel,
        mesh=mesh,
        out_type=jax.ShapeDtypeStruct((_NROWS, 1, 128), F32),
        scratch_types=[pltpu.VMEM((_BPW,), jnp.int32),
                       pltpu.VMEM((_BPW, 1, 128), F32),
                       pltpu.SemaphoreType.DMA],
    )(_sc_gather_body)
    return f(table3, idx_flat)


def kernel(x, params):
    B = x.shape[0]
    xn = jnp.transpose(x, (0, 2, 3, 1))
    xp = jnp.pad(xn, ((0, 0), (1, 1), (1, 1), (0, 0)))

    taps = [xp[:, kh:kh + 224:2, kw:kw + 224:2, :]
            for kh in range(4) for kw in range(4)]
    patches = jnp.concatenate(taps, axis=-1).reshape(B, 112 * 112, 48)
    w1 = jnp.transpose(params['down_w'][0], (2, 3, 1, 0)).reshape(48, 128)
    b1 = params['down_b'][0].reshape(1, 128)
    h1 = _conv1(patches, w1, b1).reshape(B, 112, 112, 128)

    h1p = jnp.pad(h1, ((0, 0), (1, 1), (1, 1), (0, 0)))
    p00 = h1p[:, 0::2, 0::2, :]
    p01 = h1p[:, 0::2, 1::2, :]
    p10 = h1p[:, 1::2, 0::2, :]
    p11 = h1p[:, 1::2, 1::2, :]
    w2 = jnp.transpose(params['down_w'][1], (2, 3, 1, 0)).reshape(16, 128, 256)
    b2 = params['down_b'][1].reshape(1, 256)
    h2 = _conv2(p00, p01, p10, p11, w2, b2)

    rw1 = [jnp.transpose(w, (2, 3, 1, 0)).reshape(9, 256, 256)
           for w in params['res_w1']]
    rw2 = [jnp.transpose(w[:, :, 0, 0], (1, 0)) for w in params['res_w2']]
    h2p = jnp.pad(h2, ((0, 0), (1, 1), (1, 1), (0, 0)))
    h3 = _resblock(h2p, rw1[0], rw2[0])

    emb = params['codebook']
    wo = jnp.transpose(params['out_w'][:, :, 0, 0], (1, 0))
    bo = params['out_b'].reshape(1, 64)
    embT = jnp.transpose(emb, (1, 0))
    e2 = jnp.sum(emb * emb, axis=1).reshape(1, 512)
    h3p = jnp.pad(h3, ((0, 0), (1, 1), (1, 1), (0, 0)))
    lat, idx = _res_proj_vq(h3p, rw1[1], rw2[1], wo, bo, embT, e2)

    emb_pad = jnp.pad(emb, ((0, 0), (0, 64))).reshape(512, 1, 128)
    q = _sc_gather(emb_pad, idx.reshape(_NROWS))[:, 0, :64].reshape(B, 3136, 64)

    latent = jnp.transpose(lat.reshape(B, 56, 56, 64), (0, 3, 1, 2))
    quantized = jnp.transpose(q.reshape(B, 56, 56, 64), (0, 3, 1, 2))
    indices = idx.reshape(B, 56, 56)
    return latent, quantized, quantized, indices


# speedup vs baseline: 1.3131x; 1.3114x over previous
import jax
import jax.numpy as jnp
from jax import lax
from jax.experimental import pallas as pl

F32 = jnp.float32


def _mm(a, b):
    return lax.dot_general(a, b, (((1,), (0,)), ((), ())),
                           preferred_element_type=F32)


def _k1_body(p_ref, w_ref, b_ref, o_ref):
    y = _mm(p_ref[0], w_ref[...]) + b_ref[...]
    o_ref[0] = jnp.maximum(y, 0.0)


def _conv1(patches, w, b):
    B, N, K = patches.shape
    G = 4
    Nb = N // G
    return pl.pallas_call(
        _k1_body,
        grid=(B, G),
        in_specs=[pl.BlockSpec((1, Nb, K), lambda bb, g: (bb, g, 0)),
                  pl.BlockSpec((K, 128), lambda bb, g: (0, 0)),
                  pl.BlockSpec((1, 128), lambda bb, g: (0, 0))],
        out_specs=pl.BlockSpec((1, Nb, 128), lambda bb, g: (bb, g, 0)),
        out_shape=jax.ShapeDtypeStruct((B, N, 128), F32),
    )(patches, w, b)


def _k2_body(p00_ref, p01_ref, p10_ref, p11_ref, w_ref, b_ref, o_ref):
    g = pl.program_id(1)
    P = ((p00_ref, p01_ref), (p10_ref, p11_ref))
    acc = jnp.zeros((14 * 56, 256), F32)
    for kh in range(4):
        a, r0 = kh % 2, kh // 2
        for kw in range(4):
            c, c0 = kw % 2, kw // 2
            slab = P[a][c][0, pl.ds(g * 14 + r0, 14), pl.ds(c0, 56), :]
            acc += _mm(slab.reshape(14 * 56, 128), w_ref[kh * 4 + kw])
    o_ref[0] = jnp.maximum(acc + b_ref[...], 0.0).reshape(14, 56, 256)


def _conv2(p00, p01, p10, p11, w, b):
    return pl.pallas_call(
        _k2_body,
        grid=(8, 4),
        in_specs=[pl.BlockSpec((1, 57, 57, 128), lambda bb, g: (bb, 0, 0, 0))] * 4
        + [pl.BlockSpec((16, 128, 256), lambda bb, g: (0, 0, 0)),
           pl.BlockSpec((1, 256), lambda bb, g: (0, 0))],
        out_specs=pl.BlockSpec((1, 14, 56, 256), lambda bb, g: (bb, g, 0, 0)),
        out_shape=jax.ShapeDtypeStruct((8, 56, 56, 256), F32),
    )(p00, p01, p10, p11, w, b)


def _k3_body(x_ref, w1_ref, w2_ref, o_ref):
    g = pl.program_id(1)
    acc = jnp.zeros((784, 256), F32)
    for kh in range(3):
        for kw in range(3):
            slab = x_ref[0, pl.ds(g * 14 + kh, 14), pl.ds(kw, 56), :]
            acc += _mm(jnp.maximum(slab, 0.0).reshape(784, 256),
                       w1_ref[kh * 3 + kw])
    r = _mm(jnp.maximum(acc, 0.0), w2_ref[...])
    ctr = x_ref[0, pl.ds(g * 14 + 1, 14), pl.ds(1, 56), :].reshape(784, 256)
    o_ref[0] = (ctr + r).reshape(14, 56, 256)


def _resblock(xpad, w1, w2):
    return pl.pallas_call(
        _k3_body,
        grid=(8, 4),
        in_specs=[pl.BlockSpec((1, 58, 58, 256), lambda bb, g: (bb, 0, 0, 0)),
                  pl.BlockSpec((9, 256, 256), lambda bb, g: (0, 0, 0)),
                  pl.BlockSpec((256, 256), lambda bb, g: (0, 0))],
        out_specs=pl.BlockSpec((1, 14, 56, 256), lambda bb, g: (bb, g, 0, 0)),
        out_shape=jax.ShapeDtypeStruct((8, 56, 56, 256), F32),
    )(xpad, w1, w2)


def _k4_body(x_ref, w1_ref, w2_ref, wo_ref, bo_ref, embT_ref, e2_ref,
             emb_ref, lat_ref, q_ref, idx_ref):
    g = pl.program_id(1)
    acc = jnp.zeros((784, 256), F32)
    for kh in range(3):
        for kw in range(3):
            slab = x_ref[0, pl.ds(g * 14 + kh, 14), pl.ds(kw, 56), :]
            acc += _mm(jnp.maximum(slab, 0.0).reshape(784, 256),
                       w1_ref[kh * 3 + kw])
    r = _mm(jnp.maximum(acc, 0.0), w2_ref[...])
    ctr = x_ref[0, pl.ds(g * 14 + 1, 14), pl.ds(1, 56), :].reshape(784, 256)
    h = ctr + r
    f = _mm(jnp.maximum(h, 0.0), wo_ref[...]) + bo_ref[...]
    lat_ref[0] = f
    f2 = jnp.sum(f * f, axis=1, keepdims=True)
    s = (f2 + e2_ref[...]) - 2.0 * _mm(f, embT_ref[...])
    m = jnp.min(s, axis=1, keepdims=True)
    iota = lax.broadcasted_iota(jnp.int32, (784, 512), 1)
    idxv = jnp.min(jnp.where(s == m, iota, 512), axis=1)
    idx_ref[0, 0, 0] = idxv
    oh = (iota == idxv[:, None]).astype(F32)
    q_ref[0] = _mm(oh, emb_ref[...])


def _res_proj_vq(xpad, w1, w2, wo, bo, embT, e2, emb):
    return pl.pallas_call(
        _k4_body,
        grid=(8, 4),
        in_specs=[pl.BlockSpec((1, 58, 58, 256), lambda bb, g: (bb, 0, 0, 0)),
                  pl.BlockSpec((9, 256, 256), lambda bb, g: (0, 0, 0)),
                  pl.BlockSpec((256, 256), lambda bb, g: (0, 0)),
                  pl.BlockSpec((256, 64), lambda bb, g: (0, 0)),
                  pl.BlockSpec((1, 64), lambda bb, g: (0, 0)),
                  pl.BlockSpec((64, 512), lambda bb, g: (0, 0)),
                  pl.BlockSpec((1, 512), lambda bb, g: (0, 0)),
                  pl.BlockSpec((512, 64), lambda bb, g: (0, 0))],
        out_specs=[pl.BlockSpec((1, 784, 64), lambda bb, g: (bb, g, 0)),
                   pl.BlockSpec((1, 784, 64), lambda bb, g: (bb, g, 0)),
                   pl.BlockSpec((1, 1, 1, 784), lambda bb, g: (bb, g, 0, 0))],
        out_shape=[jax.ShapeDtypeStruct((8, 3136, 64), F32),
                   jax.ShapeDtypeStruct((8, 3136, 64), F32),
                   jax.ShapeDtypeStruct((8, 4, 1, 784), jnp.int32)],
    )(xpad, w1, w2, wo, bo, embT, e2, emb)


def kernel(x, params):
    B = x.shape[0]
    xn = jnp.transpose(x, (0, 2, 3, 1))
    xp = jnp.pad(xn, ((0, 0), (1, 1), (1, 1), (0, 0)))

    taps = [xp[:, kh:kh + 224:2, kw:kw + 224:2, :]
            for kh in range(4) for kw in range(4)]
    patches = jnp.concatenate(taps, axis=-1).reshape(B, 112 * 112, 48)
    w1 = jnp.transpose(params['down_w'][0], (2, 3, 1, 0)).reshape(48, 128)
    b1 = params['down_b'][0].reshape(1, 128)
    h1 = _conv1(patches, w1, b1).reshape(B, 112, 112, 128)

    h1p = jnp.pad(h1, ((0, 0), (1, 1), (1, 1), (0, 0)))
    p00 = h1p[:, 0::2, 0::2, :]
    p01 = h1p[:, 0::2, 1::2, :]
    p10 = h1p[:, 1::2, 0::2, :]
    p11 = h1p[:, 1::2, 1::2, :]
    w2 = jnp.transpose(params['down_w'][1], (2, 3, 1, 0)).reshape(16, 128, 256)
    b2 = params['down_b'][1].reshape(1, 256)
    h2 = _conv2(p00, p01, p10, p11, w2, b2)

    rw1 = [jnp.transpose(w, (2, 3, 1, 0)).reshape(9, 256, 256)
           for w in params['res_w1']]
    rw2 = [jnp.transpose(w[:, :, 0, 0], (1, 0)) for w in params['res_w2']]
    h2p = jnp.pad(h2, ((0, 0), (1, 1), (1, 1), (0, 0)))
    h3 = _resblock(h2p, rw1[0], rw2[0])

    emb = params['codebook']
    wo = jnp.transpose(params['out_w'][:, :, 0, 0], (1, 0))
    bo = params['out_b'].reshape(1, 64)
    embT = jnp.transpose(emb, (1, 0))
    e2 = jnp.sum(emb * emb, axis=1).reshape(1, 512)
    h3p = jnp.pad(h3, ((0, 0), (1, 1), (1, 1), (0, 0)))
    lat, q, idx = _res_proj_vq(h3p, rw1[1], rw2[1], wo, bo, embT, e2, emb)

    latent = jnp.transpose(lat.reshape(B, 56, 56, 64), (0, 3, 1, 2))
    quantized = jnp.transpose(q.reshape(B, 56, 56, 64), (0, 3, 1, 2))
    indices = idx.reshape(B, 56, 56)
    return latent, quantized, quantized, indices
